# baseline (speedup 1.0000x reference)
import functools

import jax
import jax.numpy as jnp
from jax import lax
from jax.experimental import pallas as pl
from jax.experimental.pallas import tpu as pltpu
from jax.experimental.pallas import tpu_sc as plsc

B = 16
N = 512
K = 128
C = 128
CH = 16

_NC = 2
_NS = 16
_NW = _NC * _NS

_EDGE_ROWS = B * K * K
_EPW = _EDGE_ROWS // _NW
_ROUNDS = 4
_RCHUNK = _EPW // _ROUNDS
_NGATH = _RCHUNK // 128
_NODE_ROWS = B * K
_NPW = _NODE_ROWS // _NW


def _topk_body(sim_ref, nfidx_ref, efidx_ref):
    b = pl.program_id(0)
    s = sim_ref[...].reshape(1, N)
    sj = jnp.broadcast_to(s, (N, N))
    si = sj.T
    ii = lax.broadcasted_iota(jnp.int32, (N, N), 0)
    jj = lax.broadcasted_iota(jnp.int32, (N, N), 1)
    prec = ((si < sj) | ((si == sj) & (ii < jj))).astype(jnp.float32)
    rank = jnp.sum(prec, axis=0, keepdims=True)
    sel = (rank < float(K)).astype(jnp.float32)
    tri = (ii < jj).astype(jnp.float32)
    pos = lax.dot_general(sel, tri, (((1,), (0,)), ((), ())),
                          preferred_element_type=jnp.float32)
    i128 = lax.broadcasted_iota(jnp.int32, (K, N), 0).astype(jnp.float32)
    onehot = ((jnp.broadcast_to(pos, (K, N)) == i128)
              & (jnp.broadcast_to(sel, (K, N)) > 0.0)).astype(jnp.float32)
    j512 = lax.broadcasted_iota(jnp.int32, (K, N), 1).astype(jnp.float32)
    idx_col = jnp.sum(onehot * j512, axis=1, keepdims=True)
    jr = lax.broadcasted_iota(jnp.int32, (1, N), 1)
    jhi = (jr // 2).astype(jnp.float32)
    jlo = (jr % 2).astype(jnp.float32)
    hi = lax.dot_general(jhi, onehot, (((1,), (1,)), ((), ())),
                         preferred_element_type=jnp.float32)
    lo = lax.dot_general(jlo, onehot, (((1,), (1,)), ((), ())),
                         preferred_element_type=jnp.float32)
    idx_row = 2.0 * hi + lo
    bf = b.astype(jnp.float32)
    nfidx_ref[...] = (bf * float(N) + idx_row).astype(jnp.int32).reshape(1, 1, K)
    ef = bf * float(N * N) + idx_col * float(N) + idx_row
    efidx_ref[...] = ef.astype(jnp.int32).reshape(1, K, K)


def _topk_indices(sim3):
    return pl.pallas_call(
        _topk_body,
        grid=(B,),
        in_specs=[pl.BlockSpec((1, 1, N), lambda b: (b, 0, 0))],
        out_specs=[
            pl.BlockSpec((1, 1, K), lambda b: (b, 0, 0)),
            pl.BlockSpec((1, K, K), lambda b: (b, 0, 0)),
        ],
        out_shape=[
            jax.ShapeDtypeStruct((B, 1, K), jnp.int32),
            jax.ShapeDtypeStruct((B, K, K), jnp.int32),
        ],
        compiler_params=pltpu.CompilerParams(
            dimension_semantics=("arbitrary",)),
    )(sim3)


def _sc_gather_body(ntab, nidx, etab, eidx, nodes_out, e_out,
                    nidx_v, nbuf, eidx_v, ebuf, nsem, esem):
    wid = lax.axis_index("s") * _NC + lax.axis_index("c")
    nb = wid * _NPW
    pltpu.sync_copy(nidx.at[pl.ds(nb, _NPW)], nidx_v)
    pltpu.sync_copy(eidx.at[pl.ds(nb, _NPW)], eidx_v)
    ncopy = pltpu.async_copy(ntab.at[nidx_v], nbuf, nsem)
    ncopy.wait()
    pltpu.sync_copy(nbuf, nodes_out.at[pl.ds(nb, _NPW)])

    def round_body(r, carry):
        handles = []
        for q in range(_NGATH):
            h = pltpu.async_copy(etab.at[eidx_v.at[r * _NGATH + q]],
                                 ebuf.at[pl.ds(q * 128, 128)], esem)
            handles.append(h)
        for h in handles:
            h.wait()
        pltpu.sync_copy(ebuf, e_out.at[pl.ds(wid * _EPW + r * _RCHUNK, _RCHUNK)])
        return carry

    lax.fori_loop(0, _ROUNDS, round_body, 0)


@functools.lru_cache(maxsize=1)
def _sc_gather_fn():
    mesh = plsc.VectorSubcoreMesh(core_axis_name="c", subcore_axis_name="s",
                                  num_cores=_NC, num_subcores=_NS)
    return pl.kernel(
        _sc_gather_body,
        out_type=(jax.ShapeDtypeStruct((_NODE_ROWS, C), jnp.float32),
                  jax.ShapeDtypeStruct((_EDGE_ROWS, CH), jnp.float32)),
        mesh=mesh,
        scratch_types=[
            pltpu.VMEM((_NPW,), jnp.int32),
            pltpu.VMEM((_NPW, C), jnp.float32),
            pltpu.VMEM((_NPW, 128), jnp.int32),
            pltpu.VMEM((_RCHUNK, CH), jnp.float32),
            pltpu.SemaphoreType.DMA,
            pltpu.SemaphoreType.DMA,
        ],
        compiler_params=pltpu.CompilerParams(use_tc_tiling_on_sc=False),
    )


def kernel(obj_similarity, obj_mmt_in, obj_obj_edge_feat):
    nfidx, efidx = _topk_indices(obj_similarity.reshape(B, 1, N))
    nodes_flat, e_flat = _sc_gather_fn()(
        obj_mmt_in.reshape(B * N, C),
        nfidx.reshape(_NODE_ROWS),
        obj_obj_edge_feat.reshape(B * N * N, CH),
        efidx.reshape(_NODE_ROWS, 128),
    )
    nodes = nodes_flat.reshape(B, K, C)
    e = e_flat.reshape(B, K, K, CH)
    mask = jnp.ones((B, K), dtype=jnp.float32)
    return nodes, mask, e
